# Initial kernel scaffold; baseline (speedup 1.0000x reference)
import functools

import jax
import jax.numpy as jnp
from jax import lax
from jax.experimental import pallas as pl
from jax.experimental.pallas import tpu as pltpu
from jax.experimental.pallas import tpu_sc as plsc

_NC = 2
_NS = 16
_NW = _NC * _NS
_CHUNK = 128


@functools.lru_cache(maxsize=None)
def _build(total, D):
    per_w = total // _NW
    n_chunks = per_w // _CHUNK
    mesh = plsc.VectorSubcoreMesh(core_axis_name="c", subcore_axis_name="s")

    @functools.partial(
        pl.kernel,
        mesh=mesh,
        out_type=jax.ShapeDtypeStruct((total, D), jnp.float32),
        scratch_types=[
            pltpu.VMEM((n_chunks, _CHUNK), jnp.int32),
            pltpu.VMEM((_CHUNK, D), jnp.float32),
            pltpu.SemaphoreType.DMA,
        ],
    )
    def k(table, ids3, out, idx_v, rows_v, sem):
        wid = lax.axis_index("s") * _NC + lax.axis_index("c")
        base = wid * per_w
        pltpu.sync_copy(ids3.at[wid], idx_v)

        def body(j, carry):
            pltpu.async_copy(table.at[idx_v.at[j]], rows_v, sem).wait()
            pltpu.sync_copy(rows_v, out.at[pl.ds(base + j * _CHUNK, _CHUNK)])
            return carry

        lax.fori_loop(0, n_chunks, body, 0)

    return k


def kernel(ids, emb_table):
    B, S = ids.shape
    V, D = emb_table.shape
    total = B * S
    ids3 = ids.astype(jnp.int32).reshape(_NW, total // (_NW * _CHUNK), _CHUNK)
    out = _build(total, D)(emb_table, ids3)
    return out.reshape(B, S, D)


# speedup vs baseline: 3.4432x; 3.4432x over previous
import functools

import jax
import jax.numpy as jnp
from jax import lax
from jax.experimental import pallas as pl
from jax.experimental.pallas import tpu as pltpu
from jax.experimental.pallas import tpu_sc as plsc

_NC = 2
_NS = 16
_NW = _NC * _NS
_CHUNK = 128


@functools.lru_cache(maxsize=None)
def _build(total, D):
    per_w = total // _NW
    n_chunks = per_w // _CHUNK
    mesh = plsc.VectorSubcoreMesh(core_axis_name="c", subcore_axis_name="s")

    @functools.partial(
        pl.kernel,
        mesh=mesh,
        out_type=jax.ShapeDtypeStruct((total, D), jnp.float32),
        scratch_types=[
            pltpu.VMEM((n_chunks, _CHUNK), jnp.int32),
            pltpu.VMEM((_CHUNK, D), jnp.float32),
            pltpu.SemaphoreType.DMA,
        ],
        compiler_params=pltpu.CompilerParams(use_tc_tiling_on_sc=False),
    )
    def k(table, ids3, out, idx_v, rows_v, sem):
        wid = lax.axis_index("s") * _NC + lax.axis_index("c")
        base = wid * per_w
        pltpu.sync_copy(ids3.at[wid], idx_v)

        def body(j, carry):
            pltpu.async_copy(table.at[idx_v.at[j]], rows_v, sem).wait()
            pltpu.sync_copy(rows_v, out.at[pl.ds(base + j * _CHUNK, _CHUNK)])
            return carry

        lax.fori_loop(0, n_chunks, body, 0)

    return k


def kernel(ids, emb_table):
    B, S = ids.shape
    V, D = emb_table.shape
    total = B * S
    ids3 = ids.astype(jnp.int32).reshape(_NW, total // (_NW * _CHUNK), _CHUNK)
    out = _build(total, D)(emb_table, ids3)
    return out.reshape(B, S, D)


# speedup vs baseline: 3.6005x; 1.0457x over previous
import functools

import jax
import jax.numpy as jnp
from jax import lax
from jax.experimental import pallas as pl
from jax.experimental.pallas import tpu as pltpu
from jax.experimental.pallas import tpu_sc as plsc

_NC = 2
_NS = 16
_NW = _NC * _NS
_CHUNK = 128


@functools.lru_cache(maxsize=None)
def _build(total, D):
    per_w = total // _NW
    n_chunks = per_w // _CHUNK
    mesh = plsc.VectorSubcoreMesh(core_axis_name="c", subcore_axis_name="s")

    nb = 8
    assert n_chunks % nb == 0 and n_chunks // nb >= 2
    n_groups = n_chunks // nb

    @functools.partial(
        pl.kernel,
        mesh=mesh,
        out_type=jax.ShapeDtypeStruct((total, D), jnp.float32),
        scratch_types=[
            pltpu.VMEM((n_chunks, _CHUNK), jnp.int32),
            pltpu.VMEM((nb, _CHUNK, D), jnp.float32),
            pltpu.SemaphoreType.DMA((nb,)),
        ],
        compiler_params=pltpu.CompilerParams(use_tc_tiling_on_sc=False),
    )
    def k(table, ids3, out, idx_v, rows_v, sems):
        wid = lax.axis_index("s") * _NC + lax.axis_index("c")
        base = wid * per_w
        pltpu.sync_copy(ids3.at[wid], idx_v)

        def gather_start(j, b):
            pltpu.async_copy(table.at[idx_v.at[j]], rows_v.at[b], sems.at[b])

        def gather_wait(b):
            pltpu.make_async_copy(table.at[idx_v.at[0]], rows_v.at[b],
                                  sems.at[b]).wait()

        for b in range(nb):
            gather_start(b, b)

        def group(g, carry):
            for b in range(nb):
                j = g * nb + b
                gather_wait(b)
                pltpu.sync_copy(rows_v.at[b],
                                out.at[pl.ds(base + j * _CHUNK, _CHUNK)])
                gather_start(j + nb, b)
            return carry

        lax.fori_loop(0, n_groups - 1, group, 0)

        for b in range(nb):
            j = (n_groups - 1) * nb + b
            gather_wait(b)
            pltpu.sync_copy(rows_v.at[b],
                            out.at[pl.ds(base + j * _CHUNK, _CHUNK)])

    return k


def kernel(ids, emb_table):
    B, S = ids.shape
    V, D = emb_table.shape
    total = B * S
    ids3 = ids.astype(jnp.int32).reshape(_NW, total // (_NW * _CHUNK), _CHUNK)
    out = _build(total, D)(emb_table, ids3)
    return out.reshape(B, S, D)
